# baseline (speedup 1.0000x reference)
import jax
import jax.numpy as jnp
from jax.experimental import pallas as pl
from jax.experimental.pallas import tpu as pltpu

EMBED_DIM = 16


def _body(idx_ref, table_ref, out_ref, blk_v, sem):
    r = idx_ref[0]
    base = pl.multiple_of((r // 128) * 128, 128)
    cp = pltpu.make_async_copy(table_ref.at[:, pl.ds(base, 128)], blk_v, sem)
    cp.start()
    cp.wait()
    rolled = pltpu.roll(blk_v[...], -(r % 128), 1)
    out_ref[...] = jnp.swapaxes(rolled[:, :1], 0, 1)


def kernel(client_id, embed_table):
    idx = jnp.asarray(client_id, dtype=jnp.int32).reshape((1,))
    return pl.pallas_call(
        _body,
        in_specs=[
            pl.BlockSpec(memory_space=pltpu.SMEM),
            pl.BlockSpec(memory_space=pl.ANY),
        ],
        out_shape=jax.ShapeDtypeStruct((1, EMBED_DIM), jnp.float32),
        scratch_shapes=[
            pltpu.VMEM((EMBED_DIM, 128), jnp.float32),
            pltpu.SemaphoreType.DMA,
        ],
    )(idx, embed_table.T)


# speedup vs baseline: 1.0223x; 1.0223x over previous
import jax
import jax.numpy as jnp
from jax.experimental import pallas as pl
from jax.experimental.pallas import tpu as pltpu

EMBED_DIM = 16


def _body(idx_ref, table_ref, out_ref, blk_v, sem):
    r = idx_ref[...]
    base = pl.multiple_of((r // 128) * 128, 128)
    cp = pltpu.make_async_copy(table_ref.at[:, pl.ds(base, 128)], blk_v, sem)
    cp.start()
    cp.wait()
    rolled = pltpu.roll(blk_v[...], -(r % 128), 1)
    out_ref[...] = jnp.swapaxes(rolled[:, :1], 0, 1)


def kernel(client_id, embed_table):
    idx = jnp.asarray(client_id, dtype=jnp.int32)
    return pl.pallas_call(
        _body,
        in_specs=[
            pl.BlockSpec(memory_space=pltpu.SMEM),
            pl.BlockSpec(memory_space=pl.ANY),
        ],
        out_shape=jax.ShapeDtypeStruct((1, EMBED_DIM), jnp.float32),
        scratch_shapes=[
            pltpu.VMEM((EMBED_DIM, 128), jnp.float32),
            pltpu.SemaphoreType.DMA,
        ],
    )(idx, embed_table.T)
